# baseline (speedup 1.0000x reference)
import functools

import jax
import jax.numpy as jnp
from jax.experimental import pallas as pl

_EPS = 1e-5
_BF = jnp.bfloat16


def _pick_block(n, candidates):
    for c in candidates:
        if n % c == 0:
            return c
    return n


def _tree_accum(acc, x):
    rows, hid = x.shape
    return acc + jnp.sum(x.reshape(rows // 8, 8, hid), axis=0)


def _finish_stat(s, inv_n):
    a = s[0:4, :] + s[4:8, :]
    b = a[0:2, :] + a[2:4, :]
    c = b[0:1, :] + b[1:2, :]
    return c * inv_n


def _bn_apply(t, mean, var, g, be):
    return jnp.maximum((t - mean) / jnp.sqrt(var + _EPS) * g + be, 0.0)


def _k1_body(adj_ref, x_ref, w1_ref, b1_ref, t_ref, s_ref):
    m = pl.program_id(0)
    pooled = jnp.dot(adj_ref[...].astype(_BF), x_ref[...],
                     preferred_element_type=jnp.float32)
    t = jnp.dot(pooled.astype(_BF), w1_ref[...],
                preferred_element_type=jnp.float32) + b1_ref[...]
    t_ref[...] = t

    @pl.when(m == 0)
    def _init():
        s_ref[...] = jnp.zeros_like(s_ref)

    s_ref[...] = _tree_accum(s_ref[...], t)


def _k1(adj, x, w1, b1, bm):
    n, k = adj.shape
    din = x.shape[1]
    hid = w1.shape[1]
    return pl.pallas_call(
        _k1_body,
        grid=(n // bm,),
        in_specs=[
            pl.BlockSpec((bm, k), lambda m: (m, 0)),
            pl.BlockSpec((k, din), lambda m: (0, 0)),
            pl.BlockSpec((din, hid), lambda m: (0, 0)),
            pl.BlockSpec((1, hid), lambda m: (0, 0)),
        ],
        out_specs=[
            pl.BlockSpec((bm, hid), lambda m: (m, 0)),
            pl.BlockSpec((8, hid), lambda m: (0, 0)),
        ],
        out_shape=[
            jax.ShapeDtypeStruct((n, hid), jnp.float32),
            jax.ShapeDtypeStruct((8, hid), jnp.float32),
        ],
    )(adj, x, w1, b1)


def _kv_body(inv_n, t_ref, s_ref, ssq_ref):
    m = pl.program_id(0)
    mean = _finish_stat(s_ref[...], inv_n)
    d = t_ref[...] - mean

    @pl.when(m == 0)
    def _init():
        ssq_ref[...] = jnp.zeros_like(ssq_ref)

    ssq_ref[...] = _tree_accum(ssq_ref[...], d * d)


def _kv(t, s, bm):
    n, hid = t.shape
    return pl.pallas_call(
        functools.partial(_kv_body, 1.0 / n),
        grid=(n // bm,),
        in_specs=[
            pl.BlockSpec((bm, hid), lambda m: (m, 0)),
            pl.BlockSpec((8, hid), lambda m: (0, 0)),
        ],
        out_specs=pl.BlockSpec((8, hid), lambda m: (0, 0)),
        out_shape=jax.ShapeDtypeStruct((8, hid), jnp.float32),
    )(t, s)


def _k2_body(inv_n, t_ref, s_ref, ssq_ref, g_ref, be_ref, w2_ref, b2_ref,
             u_ref, s2_ref):
    m = pl.program_id(0)
    mean = _finish_stat(s_ref[...], inv_n)
    var = _finish_stat(ssq_ref[...], inv_n)
    h = _bn_apply(t_ref[...], mean, var, g_ref[...], be_ref[...])
    u = jnp.dot(h.astype(_BF), w2_ref[...],
                preferred_element_type=jnp.float32) + b2_ref[...]
    u_ref[...] = u

    @pl.when(m == 0)
    def _init():
        s2_ref[...] = jnp.zeros_like(s2_ref)

    s2_ref[...] = _tree_accum(s2_ref[...], u)


def _k2(t, s, ssq, g, be, w2, b2, bm):
    n, hid = t.shape
    hid2 = w2.shape[1]
    return pl.pallas_call(
        functools.partial(_k2_body, 1.0 / n),
        grid=(n // bm,),
        in_specs=[
            pl.BlockSpec((bm, hid), lambda m: (m, 0)),
            pl.BlockSpec((8, hid), lambda m: (0, 0)),
            pl.BlockSpec((8, hid), lambda m: (0, 0)),
            pl.BlockSpec((1, hid), lambda m: (0, 0)),
            pl.BlockSpec((1, hid), lambda m: (0, 0)),
            pl.BlockSpec((hid, hid2), lambda m: (0, 0)),
            pl.BlockSpec((1, hid2), lambda m: (0, 0)),
        ],
        out_specs=[
            pl.BlockSpec((bm, hid2), lambda m: (m, 0)),
            pl.BlockSpec((8, hid2), lambda m: (0, 0)),
        ],
        out_shape=[
            jax.ShapeDtypeStruct((n, hid2), jnp.float32),
            jax.ShapeDtypeStruct((8, hid2), jnp.float32),
        ],
    )(t, s, ssq, g, be, w2, b2)


def _k3_body(inv_n, u_ref, s_ref, ssq_ref, g_ref, be_ref, o_ref):
    mean = _finish_stat(s_ref[...], inv_n)
    var = _finish_stat(ssq_ref[...], inv_n)
    o = _bn_apply(u_ref[...], mean, var, g_ref[...], be_ref[...])
    o_ref[...] = o.astype(o_ref.dtype)


def _k3(u, s, ssq, g, be, bm, out_dtype=jnp.float32):
    n, hid = u.shape
    return pl.pallas_call(
        functools.partial(_k3_body, 1.0 / n),
        grid=(n // bm,),
        in_specs=[
            pl.BlockSpec((bm, hid), lambda m: (m, 0)),
            pl.BlockSpec((8, hid), lambda m: (0, 0)),
            pl.BlockSpec((8, hid), lambda m: (0, 0)),
            pl.BlockSpec((1, hid), lambda m: (0, 0)),
            pl.BlockSpec((1, hid), lambda m: (0, 0)),
        ],
        out_specs=pl.BlockSpec((bm, hid), lambda m: (m, 0)),
        out_shape=jax.ShapeDtypeStruct((n, hid), out_dtype),
    )(u, s, ssq, g, be)


def _layer(adj, x, w1, b1, g1, be1, w2, b2, bng, bnb, bm1, bm2,
           out_dtype=jnp.float32):
    t, s_t = _k1(adj, x, w1.astype(_BF), b1, bm1)
    ssq_t = _kv(t, s_t, bm2)
    u, s_u = _k2(t, s_t, ssq_t, g1, be1, w2.astype(_BF), b2, bm2)
    ssq_u = _kv(u, s_u, bm2)
    return _k3(u, s_u, ssq_u, bng, bnb, bm2, out_dtype)


def kernel(Adj, feats, W0_1, b0_1, g0_1, be0_1, W0_2, b0_2, bn0_g, bn0_b,
           W1_1, b1_1, g1_1, be1_1, W1_2, b1_2, bn1_g, bn1_b):
    n = Adj.shape[0]
    row = lambda v: v.reshape(1, -1)
    bm1 = _pick_block(n, (400, 256, 200, 128, 100, 80, 64, 40, 32, 16, 8))
    bm2 = _pick_block(n, (2000, 1024, 1000, 512, 400, 256, 200, 80, 40, 16, 8))
    h0 = _layer(Adj, feats.astype(_BF), W0_1, row(b0_1), row(g0_1),
                row(be0_1), W0_2, row(b0_2), row(bn0_g), row(bn0_b),
                bm1, bm2, out_dtype=_BF)
    h1 = _layer(Adj, h0, W1_1, row(b1_1), row(g1_1), row(be1_1),
                W1_2, row(b1_2), row(bn1_g), row(bn1_b), bm1, bm2)
    return h1
